# baseline (speedup 1.0000x reference)
import dataclasses
import functools

import jax
import jax.numpy as jnp
from jax import lax
from jax.experimental import pallas as pl
from jax.experimental.pallas import tpu as pltpu
from jax.experimental.pallas import tpu_sc as plsc

H = W = 224
WP = 232
HPAD = 240
SR = 32
NSTRIP = H // SR
OROWS = SR + 8
L1 = 8
N = 8
NBLK = (H // 8) * (W // 8)
ULEN = NBLK * 64
NUNITS = N * L1


def _codes_body(xp_ref, w1_ref, w2_ref, out_ref, o1_ref):
    sid = pl.program_id(1)
    r0 = sid * SR
    o1_ref[...] = jnp.zeros((L1, OROWS, WP), jnp.float32)
    accs = [jnp.zeros((OROWS, W), jnp.float32) for _ in range(L1)]
    for c in range(3):
        win = xp_ref[0, c, pl.ds(pl.multiple_of(r0, 8), 48), :]
        for di in range(7):
            rows = win[di:di + OROWS, :]
            for dj in range(7):
                s = rows[:, dj:dj + W]
                for l1 in range(L1):
                    accs[l1] = accs[l1] + w1_ref[l1, c, di, dj] * s
    for l1 in range(L1):
        o1_ref[l1, :, 3:3 + W] = (
            accs[l1].astype(jnp.bfloat16).astype(jnp.float32))

    @pl.when(sid == 0)
    def _zero_top():
        o1_ref[:, 0:3, :] = jnp.zeros((L1, 3, WP), jnp.float32)

    @pl.when(sid == NSTRIP - 1)
    def _zero_bot():
        o1_ref[:, SR + 3:OROWS, :] = jnp.zeros((L1, 5, WP), jnp.float32)

    r = lax.broadcasted_iota(jnp.int32, (SR, W), 0) + r0
    cc = lax.broadcasted_iota(jnp.int32, (SR, W), 1)
    bb = ((r >> 3) * (W // 8) + (cc >> 3)) << 6
    for l1 in range(L1):
        acc2s = [jnp.zeros((SR, W), jnp.float32) for _ in range(6)]
        for di in range(7):
            rows = o1_ref[l1, di:di + SR, :]
            for dj in range(7):
                s = rows[:, dj:dj + W]
                for j in range(6):
                    acc2s[j] = acc2s[j] + w2_ref[j + 2, 0, di, dj] * s
        code = bb
        for j in range(6):
            code = code + jnp.where(acc2s[j] > 0, jnp.int32(1 << j),
                                    jnp.int32(0))
        out_ref[0, l1] = code


def _codes_call(x_pad, w1, w2):
    return pl.pallas_call(
        _codes_body,
        grid=(N, NSTRIP),
        in_specs=[
            pl.BlockSpec((1, 3, HPAD, WP), lambda n, s: (n, 0, 0, 0)),
            pl.BlockSpec(memory_space=pltpu.SMEM),
            pl.BlockSpec(memory_space=pltpu.SMEM),
        ],
        out_specs=pl.BlockSpec((1, L1, SR, W), lambda n, s: (n, 0, s, 0)),
        out_shape=jax.ShapeDtypeStruct((N, L1, H, W), jnp.int32),
        scratch_shapes=[pltpu.VMEM((L1, OROWS, WP), jnp.float32)],
        compiler_params=pltpu.CompilerParams(
            dimension_semantics=("parallel", "arbitrary")),
    )(x_pad, w1, w2)


def _sc_hist_body(fidx_hbm, out_hbm, idx_ref, hist_ref):
    c = lax.axis_index("c")
    s = lax.axis_index("s")
    w = s * 2 + c
    zeros16 = jnp.zeros((16,), jnp.float32)
    ones16 = jnp.ones((16,), jnp.float32)
    stride16 = lax.iota(jnp.int32, 16) * 64
    for k in range(2):
        u = w + 32 * k
        pltpu.sync_copy(fidx_hbm.at[u], idx_ref)

        @pl.loop(0, ULEN, step=16)
        def _zero(i):
            hist_ref[pl.ds(i, 16)] = zeros16

        @pl.loop(0, NBLK, step=16)
        def _grp(b):
            base = stride16 + b * 64

            @pl.loop(0, 64)
            def _px(j):
                v = plsc.load_gather(idx_ref, [base + j])
                plsc.addupdate_scatter(hist_ref, [v], ones16)

        pltpu.sync_copy(hist_ref, out_hbm.at[u])


def _hist_call(fidx):
    cp = pltpu.CompilerParams()
    if "needs_layout_passes" in pltpu.CompilerParams.__dataclass_fields__:
        cp = dataclasses.replace(cp, needs_layout_passes=False)
    mesh = plsc.VectorSubcoreMesh(core_axis_name="c", subcore_axis_name="s")
    f = pl.kernel(
        _sc_hist_body,
        out_type=jax.ShapeDtypeStruct((NUNITS, ULEN), jnp.float32),
        mesh=mesh,
        scratch_types=[
            pltpu.VMEM((ULEN,), jnp.int32),
            pltpu.VMEM((ULEN,), jnp.float32),
        ],
        compiler_params=cp,
    )
    return f(fidx)


@jax.jit
def kernel(x, w1, w2):
    xq = x.astype(jnp.bfloat16).astype(jnp.float32)
    w1q = w1.astype(jnp.bfloat16).astype(jnp.float32)
    w2q = w2.astype(jnp.bfloat16).astype(jnp.float32)
    x_pad = jnp.pad(xq, ((0, 0), (0, 0), (6, 10), (3, 5)))
    codes = _codes_call(x_pad, w1q, w2q)
    hist = _hist_call(codes.reshape(NUNITS, ULEN))
    return hist.reshape(N, L1 * ULEN)


# speedup vs baseline: 1.0147x; 1.0147x over previous
import dataclasses
import functools

import jax
import jax.numpy as jnp
from jax import lax
from jax.experimental import pallas as pl
from jax.experimental.pallas import tpu as pltpu
from jax.experimental.pallas import tpu_sc as plsc

H = W = 224
HPAD = 240
SR = 32
NSTRIP = H // SR
OROWS = SR + 8
LW = 112
ACCW = LW + 6
L1 = 8
N = 8
NBLK = (H // 8) * (W // 8)
ULEN = NBLK * 64
NUNITS = N * L1


def _codes_body(xp_ref, w1_ref, w2_ref, out_ref, o1_ref):
    sid = pl.program_id(1)
    r0 = sid * SR
    lane_o = lax.broadcasted_iota(jnp.int32, (OROWS, 128), 1)
    for h in range(2):
        accs = [jnp.zeros((OROWS, ACCW), jnp.float32) for _ in range(L1)]
        for c in range(3):
            win = xp_ref[0, c, h, pl.ds(pl.multiple_of(r0, 8), 48), :]
            for di in range(7):
                rows = win[di:di + OROWS, :]
                for dj in range(7):
                    s = rows[:, dj:dj + ACCW]
                    for l1 in range(L1):
                        accs[l1] = accs[l1] + w1_ref[l1, c, di, dj] * s
        if h == 0:
            valid = lane_o[:, :ACCW] >= 3
        else:
            valid = lane_o[:, :ACCW] < LW + 3
        for l1 in range(L1):
            a = accs[l1].astype(jnp.bfloat16).astype(jnp.float32)
            o1_ref[h, l1, :, 0:ACCW] = jnp.where(valid, a, 0.0)

    @pl.when(sid == 0)
    def _zero_top():
        o1_ref[:, :, 0:3, :] = jnp.zeros((2, L1, 3, 128), jnp.float32)

    @pl.when(sid == NSTRIP - 1)
    def _zero_bot():
        o1_ref[:, :, SR + 3:OROWS, :] = jnp.zeros((2, L1, 5, 128),
                                                  jnp.float32)

    r = lax.broadcasted_iota(jnp.int32, (SR, LW), 0) + r0
    u = lax.broadcasted_iota(jnp.int32, (SR, LW), 1)
    for h in range(2):
        bb = ((r >> 3) * (W // 8) + (14 * h + (u >> 3))) << 6
        for l1 in range(L1):
            acc2s = [jnp.zeros((SR, LW), jnp.float32) for _ in range(6)]
            for di in range(7):
                rows = o1_ref[h, l1, di:di + SR, :]
                for dj in range(7):
                    s = rows[:, dj:dj + LW]
                    for j in range(6):
                        acc2s[j] = acc2s[j] + w2_ref[j + 2, 0, di, dj] * s
            code = bb
            for j in range(6):
                code = code + jnp.where(acc2s[j] > 0, jnp.int32(1 << j),
                                        jnp.int32(0))
            out_ref[0, l1, :, h * LW:(h + 1) * LW] = code


def _codes_call(x_pad, w1, w2):
    return pl.pallas_call(
        _codes_body,
        grid=(N, NSTRIP),
        in_specs=[
            pl.BlockSpec((1, 3, 2, HPAD, 128), lambda n, s: (n, 0, 0, 0, 0)),
            pl.BlockSpec(memory_space=pltpu.SMEM),
            pl.BlockSpec(memory_space=pltpu.SMEM),
        ],
        out_specs=pl.BlockSpec((1, L1, SR, W), lambda n, s: (n, 0, s, 0)),
        out_shape=jax.ShapeDtypeStruct((N, L1, H, W), jnp.int32),
        scratch_shapes=[pltpu.VMEM((2, L1, OROWS, 128), jnp.float32)],
        compiler_params=pltpu.CompilerParams(
            dimension_semantics=("parallel", "arbitrary")),
    )(x_pad, w1, w2)


def _sc_hist_body(fidx_hbm, out_hbm, idx_ref, hist_ref):
    c = lax.axis_index("c")
    s = lax.axis_index("s")
    w = s * 2 + c
    zeros16 = jnp.zeros((16,), jnp.float32)
    ones16 = jnp.ones((16,), jnp.float32)
    stride16 = lax.iota(jnp.int32, 16) * 64
    for k in range(2):
        u = w + 32 * k
        pltpu.sync_copy(fidx_hbm.at[u], idx_ref)

        @pl.loop(0, ULEN, step=16)
        def _zero(i):
            hist_ref[pl.ds(i, 16)] = zeros16

        @pl.loop(0, NBLK, step=16)
        def _grp(b):
            base = stride16 + b * 64

            @pl.loop(0, 64)
            def _px(j):
                v = plsc.load_gather(idx_ref, [base + j])
                plsc.addupdate_scatter(hist_ref, [v], ones16)

        pltpu.sync_copy(hist_ref, out_hbm.at[u])


def _hist_call(fidx):
    cp = pltpu.CompilerParams()
    if "needs_layout_passes" in pltpu.CompilerParams.__dataclass_fields__:
        cp = dataclasses.replace(cp, needs_layout_passes=False)
    mesh = plsc.VectorSubcoreMesh(core_axis_name="c", subcore_axis_name="s")
    f = pl.kernel(
        _sc_hist_body,
        out_type=jax.ShapeDtypeStruct((NUNITS, ULEN), jnp.float32),
        mesh=mesh,
        scratch_types=[
            pltpu.VMEM((ULEN,), jnp.int32),
            pltpu.VMEM((ULEN,), jnp.float32),
        ],
        compiler_params=cp,
    )
    return f(fidx)


@jax.jit
def kernel(x, w1, w2):
    xq = x.astype(jnp.bfloat16).astype(jnp.float32)
    w1q = w1.astype(jnp.bfloat16).astype(jnp.float32)
    w2q = w2.astype(jnp.bfloat16).astype(jnp.float32)
    xc = jnp.pad(xq, ((0, 0), (0, 0), (6, 10), (6, 10)))
    x_pad = jnp.stack([xc[..., 0:128], xc[..., 112:240]], axis=2)
    codes = _codes_call(x_pad, w1q, w2q)
    hist = _hist_call(codes.reshape(NUNITS, ULEN))
    return hist.reshape(N, L1 * ULEN)
